# baseline (speedup 1.0000x reference)
import functools

import jax
import jax.numpy as jnp
from jax import lax
from jax.experimental import pallas as pl
from jax.experimental.pallas import tpu as pltpu
from jax.experimental.pallas import tpu_sc as plsc

_NF = 256
_NL = 100000
_NC = 5000
_NTOT = _NL + _NC
_S = 10.0
_B = 256
_IGN = 5555
_T = 8192
_NFULL = _NL // _T
_REM = _NL - _NFULL * _T
_GRID = _NFULL + 1


_info = plsc.get_sparse_core_info()
_NW = _info.num_cores * _info.num_subcores
_BPW = _B // _NW

_sc_mesh = plsc.VectorSubcoreMesh(core_axis_name="c", subcore_axis_name="s")


@functools.partial(
    pl.kernel,
    mesh=_sc_mesh,
    out_type=jax.ShapeDtypeStruct((_B, _NF), jnp.float32),
    scratch_types=[
        pltpu.VMEM((_BPW,), jnp.int32),
        pltpu.VMEM((_BPW, _NF), jnp.float32),
        pltpu.SemaphoreType.DMA,
    ],
)
def _sc_gather(lut_hbm, tgt_hbm, out_hbm, idx_v, rows_v, sem):
    wid = lax.axis_index("s") * _info.num_cores + lax.axis_index("c")
    base = wid * _BPW
    pltpu.sync_copy(tgt_hbm.at[pl.ds(base, _BPW)], idx_v)
    pltpu.async_copy(lut_hbm.at[idx_v], rows_v, sem).wait()
    pltpu.sync_copy(rows_v, out_hbm.at[pl.ds(base, _BPW)])


def _dot(x, w):
    return jax.lax.dot_general(
        x, w, (((1,), (1,)), ((), ())),
        preferred_element_type=jnp.float32) * _S


def _main_body(x_ref, lut_ref, cq_hbm, rows_ref, tgt_ref, out_ref, loss_ref,
               acc_ref, cq_buf, cq_sem):
    i = pl.program_id(0)

    def _cq_copy():
        return pltpu.make_async_copy(cq_hbm, cq_buf, cq_sem)

    @pl.when(i == 0)
    def _init():
        acc_ref[...] = jnp.zeros_like(acc_ref)
        _cq_copy().start()

    x = x_ref[...]

    @pl.when(i < _NFULL)
    def _lut_step():
        t = _dot(x, lut_ref[...])
        out_ref[...] = t
        acc_ref[...] += jnp.sum(jnp.exp(t - _S), axis=1, keepdims=True)

    @pl.when(i == _NFULL)
    def _last_step():
        _cq_copy().wait()
        t1 = _dot(x, lut_ref[...])
        t2 = _dot(x, cq_buf[...])
        out_ref[:, :_REM] = t1[:, :_REM]
        out_ref[:, _REM:_REM + _NC] = t2
        cols = jax.lax.broadcasted_iota(jnp.int32, (_B, _T), 1)
        e1 = jnp.where(cols < _REM, jnp.exp(t1 - _S), 0.0)
        s = (acc_ref[...]
             + jnp.sum(e1, axis=1, keepdims=True)
             + jnp.sum(jnp.exp(t2 - _S), axis=1, keepdims=True))
        g = _S * jnp.sum(x * rows_ref[...], axis=1, keepdims=True)
        lse = _S + jnp.log(s)
        nll = lse - g
        tgt = tgt_ref[...]
        tgtc = jnp.clip(tgt, 0, _NTOT - 1)
        w_cls = (tgtc < _NL).astype(jnp.float32)
        vmask = (tgt != _IGN).astype(jnp.float32)
        wgt = w_cls * vmask
        num = jnp.sum(nll * wgt)
        den = jnp.maximum(jnp.sum(wgt), 1.0)
        loss_ref[0, 0] = num / den


def kernel(inputs, targets, lut, cq):
    rows = _sc_gather(lut, targets)
    out, loss = pl.pallas_call(
        _main_body,
        grid=(_GRID,),
        in_specs=[
            pl.BlockSpec((_B, _NF), lambda i: (0, 0)),
            pl.BlockSpec((_T, _NF), lambda i: (i, 0)),
            pl.BlockSpec(memory_space=pltpu.MemorySpace.HBM),
            pl.BlockSpec((_B, _NF), lambda i: (0, 0)),
            pl.BlockSpec((_B, 1), lambda i: (0, 0)),
        ],
        out_specs=[
            pl.BlockSpec((_B, _T), lambda i: (0, i)),
            pl.BlockSpec(memory_space=pltpu.SMEM),
        ],
        out_shape=[
            jax.ShapeDtypeStruct((_B, _NTOT), jnp.float32),
            jax.ShapeDtypeStruct((1, 1), jnp.float32),
        ],
        scratch_shapes=[
            pltpu.VMEM((_B, 1), jnp.float32),
            pltpu.VMEM((_NC, _NF), jnp.float32),
            pltpu.SemaphoreType.DMA,
        ],
        compiler_params=pltpu.CompilerParams(
            dimension_semantics=("arbitrary",),
        ),
    )(inputs, lut, cq, rows, targets.reshape(_B, 1))
    return loss[0, 0], out


# speedup vs baseline: 1.0114x; 1.0114x over previous
import functools

import jax
import jax.numpy as jnp
from jax import lax
from jax.experimental import pallas as pl
from jax.experimental.pallas import tpu as pltpu
from jax.experimental.pallas import tpu_sc as plsc

_NF = 256
_NL = 100000
_NC = 5000
_NTOT = _NL + _NC
_S = 10.0
_B = 256
_IGN = 5555
_T = 8192
_NFULL = _NL // _T
_REM = _NL - _NFULL * _T
_GRID = _NFULL + 1


_info = plsc.get_sparse_core_info()
_NW = _info.num_cores * _info.num_subcores
_BPW = _B // _NW

_sc_mesh = plsc.VectorSubcoreMesh(core_axis_name="c", subcore_axis_name="s")


@functools.partial(
    pl.kernel,
    mesh=_sc_mesh,
    out_type=jax.ShapeDtypeStruct((_B, _NF), jnp.float32),
    scratch_types=[
        pltpu.VMEM((_BPW,), jnp.int32),
        pltpu.VMEM((_BPW, _NF), jnp.float32),
        pltpu.SemaphoreType.DMA,
    ],
)
def _sc_gather(lut_hbm, tgt_hbm, out_hbm, idx_v, rows_v, sem):
    wid = lax.axis_index("s") * _info.num_cores + lax.axis_index("c")
    base = wid * _BPW
    pltpu.sync_copy(tgt_hbm.at[pl.ds(base, _BPW)], idx_v)
    pltpu.async_copy(lut_hbm.at[idx_v], rows_v, sem).wait()
    pltpu.sync_copy(rows_v, out_hbm.at[pl.ds(base, _BPW)])


def _dot(x, w):
    return jax.lax.dot_general(
        x, w, (((1,), (1,)), ((), ())),
        preferred_element_type=jnp.float32) * _S


def _main_body(x_ref, lut_ref, cq_hbm, out_ref, s_ref,
               acc_ref, cq_buf, cq_sem):
    i = pl.program_id(0)

    def _cq_copy():
        return pltpu.make_async_copy(cq_hbm, cq_buf, cq_sem)

    @pl.when(i == 0)
    def _init():
        acc_ref[...] = jnp.zeros_like(acc_ref)
        _cq_copy().start()

    x = x_ref[...]

    @pl.when(i < _NFULL)
    def _lut_step():
        t = _dot(x, lut_ref[...])
        out_ref[...] = t
        acc_ref[...] += jnp.sum(jnp.exp(t - _S), axis=1, keepdims=True)

    @pl.when(i == _NFULL)
    def _last_step():
        _cq_copy().wait()
        t1 = _dot(x, lut_ref[...])
        t2 = _dot(x, cq_buf[...])
        out_ref[:, :_REM] = t1[:, :_REM]
        out_ref[:, _REM:_REM + _NC] = t2
        cols = jax.lax.broadcasted_iota(jnp.int32, (_B, _T), 1)
        e1 = jnp.where(cols < _REM, jnp.exp(t1 - _S), 0.0)
        acc = (acc_ref[...]
               + jnp.sum(e1, axis=1, keepdims=True)
               + jnp.sum(jnp.exp(t2 - _S), axis=1, keepdims=True))
        s_ref[...] = jnp.broadcast_to(acc, (_B, 128))


def _loss_body(x_ref, rows_ref, s_ref, tgt_ref, loss_ref):
    g = _S * jnp.sum(x_ref[...] * rows_ref[...], axis=1, keepdims=True)
    s = jnp.sum(s_ref[...], axis=1, keepdims=True) * (1.0 / 128.0)
    lse = _S + jnp.log(s)
    nll = lse - g
    tgt = tgt_ref[...]
    tgtc = jnp.clip(tgt, 0, _NTOT - 1)
    w_cls = (tgtc < _NL).astype(jnp.float32)
    vmask = (tgt != _IGN).astype(jnp.float32)
    wgt = w_cls * vmask
    num = jnp.sum(nll * wgt)
    den = jnp.maximum(jnp.sum(wgt), 1.0)
    loss_ref[0, 0] = num / den


def kernel(inputs, targets, lut, cq):
    rows = _sc_gather(lut, targets)
    out, s = pl.pallas_call(
        _main_body,
        grid=(_GRID,),
        in_specs=[
            pl.BlockSpec((_B, _NF), lambda i: (0, 0)),
            pl.BlockSpec((_T, _NF), lambda i: (i, 0)),
            pl.BlockSpec(memory_space=pltpu.MemorySpace.HBM),
        ],
        out_specs=[
            pl.BlockSpec((_B, _T), lambda i: (0, i)),
            pl.BlockSpec((_B, 128), lambda i: (0, 0)),
        ],
        out_shape=[
            jax.ShapeDtypeStruct((_B, _NTOT), jnp.float32),
            jax.ShapeDtypeStruct((_B, 128), jnp.float32),
        ],
        scratch_shapes=[
            pltpu.VMEM((_B, 1), jnp.float32),
            pltpu.VMEM((_NC, _NF), jnp.float32),
            pltpu.SemaphoreType.DMA,
        ],
        compiler_params=pltpu.CompilerParams(
            dimension_semantics=("arbitrary",),
        ),
    )(inputs, lut, cq)
    loss = pl.pallas_call(
        _loss_body,
        out_shape=jax.ShapeDtypeStruct((1, 1), jnp.float32),
        out_specs=pl.BlockSpec(memory_space=pltpu.SMEM),
    )(inputs, rows, s, targets.reshape(_B, 1))
    return loss[0, 0], out
